# baseline (speedup 1.0000x reference)
import functools
import jax
import jax.numpy as jnp
from jax import lax
from jax.experimental import pallas as pl
from jax.experimental.pallas import tpu as pltpu
from jax.experimental.pallas import tpu_sc as plsc

N = 10000
E = 320000
D = 128

NC = 2
NS = 16
NW = NC * NS
EPT = E // NW
CHUNK = 80
NCHUNK = EPT // CHUNK
NP = 10240
RPT = NP // NS
ZR = RPT // 5
DEGP = 10240
DPT = DEGP // NS


@functools.cache
def _build_sc_degree():
  mesh = plsc.VectorSubcoreMesh(core_axis_name="c", subcore_axis_name="s")

  @functools.partial(
      pl.kernel,
      out_type=jax.ShapeDtypeStruct((NC, DEGP), jnp.float32),
      mesh=mesh,
      scratch_types=[
          pltpu.MemorySpace.VMEM_SHARED((DEGP,), jnp.float32),
          pltpu.MemorySpace.VMEM((DPT,), jnp.float32),
          [pltpu.MemorySpace.VMEM((CHUNK,), jnp.int32) for _ in range(8)],
          [pltpu.MemorySpace.VMEM((CHUNK,), jnp.float32) for _ in range(8)],
          [pltpu.SemaphoreType.DMA for _ in range(8)],
          [pltpu.SemaphoreType.DMA for _ in range(4)],
      ],
  )
  def sc_degree(dst_hbm, ew_hbm, deg_out, dacc, zv, dst_v, ew_v, isem, ssem):
    c = lax.axis_index("c")
    s = lax.axis_index("s")
    wid = c * NS + s
    base = wid * EPT

    def ifetch(j, k8):
      off = base + j * CHUNK
      pltpu.async_copy(dst_hbm.at[pl.ds(off, CHUNK)], dst_v[k8], isem[k8])
      pltpu.async_copy(ew_hbm.at[pl.ds(off, CHUNK)], ew_v[k8], isem[k8])

    def iwait(k8):
      pltpu.make_async_copy(dst_hbm.at[pl.ds(0, CHUNK)], dst_v[k8], isem[k8]).wait()
      pltpu.make_async_copy(ew_hbm.at[pl.ds(0, CHUNK)], ew_v[k8], isem[k8]).wait()

    def swait(k4):
      pltpu.make_async_copy(ew_v[0], dacc.at[dst_v[0]], ssem[k4]).wait()

    for k in range(6):
      ifetch(k, k)

    for j in range(DPT // 16):
      zv[pl.ds(j * 16, 16)] = jnp.zeros((16,), jnp.float32)
    pltpu.sync_copy(zv, dacc.at[pl.ds(s * DPT, DPT)])
    plsc.subcore_barrier()

    def emit(j, k4, k8, has_swait, has_fetch):
      iwait(k8)
      pltpu.async_copy(ew_v[k8], dacc.at[dst_v[k8]], ssem[k4], add=True)
      if has_swait:
        swait((k4 + 2) % 4)
      if has_fetch:
        ifetch(j + 6, (k8 + 6) % 8)

    for j in range(8):
      emit(j, j % 4, j % 8, j >= 2, True)

    def body(q, carry):
      j0 = 8 * q
      for k in range(8):
        emit(j0 + k, k % 4, k, True, True)
      return carry

    lax.fori_loop(1, 14, body, 0)
    for j in range(112, NCHUNK):
      emit(j, j % 4, j % 8, True, j + 6 < NCHUNK)
    swait((NCHUNK - 2) % 4)
    swait((NCHUNK - 1) % 4)

    plsc.subcore_barrier()
    pltpu.sync_copy(dacc.at[pl.ds(s * DPT, DPT)],
                    deg_out.at[c, pl.ds(s * DPT, DPT)])

  return sc_degree


CHUNKA = 112
EPTP = 10080
NCHP = EPTP // CHUNKA


@functools.cache
def _build_sc_agg():
  mesh = plsc.VectorSubcoreMesh(core_axis_name="c", subcore_axis_name="s")

  @functools.partial(
      pl.kernel,
      out_type=jax.ShapeDtypeStruct((NC, NP, D), jnp.float32),
      mesh=mesh,
      scratch_types=[
          pltpu.MemorySpace.VMEM_SHARED((NP, D), jnp.float32),
          [pltpu.MemorySpace.VMEM((CHUNKA, D), jnp.float32) for _ in range(3)],
          [pltpu.MemorySpace.VMEM((CHUNKA,), jnp.int32) for _ in range(6)],
          [pltpu.MemorySpace.VMEM((CHUNKA,), jnp.int32) for _ in range(6)],
          [pltpu.MemorySpace.VMEM((CHUNKA,), jnp.float32) for _ in range(6)],
          [pltpu.SemaphoreType.DMA for _ in range(3)],
          [pltpu.SemaphoreType.DMA for _ in range(3)],
          [pltpu.SemaphoreType.DMA for _ in range(6)],
      ],
  )
  def sc_agg(g_hbm, src_hbm, dst_hbm, ew_hbm, part_out,
             acc, rows, src_v, dst_v, ew_v, gsem, ssem, isem):
    c = lax.axis_index("c")
    s = lax.axis_index("s")
    wid = c * NS + s
    base = wid * EPTP

    def ifetch(j, k6):
      off = base + jnp.minimum(j, NCHP - 1) * CHUNKA
      pltpu.async_copy(src_hbm.at[pl.ds(off, CHUNKA)], src_v[k6], isem[k6])
      pltpu.async_copy(dst_hbm.at[pl.ds(off, CHUNKA)], dst_v[k6], isem[k6])
      pltpu.async_copy(ew_hbm.at[pl.ds(off, CHUNKA)], ew_v[k6], isem[k6])

    def iwait(k6):
      pltpu.make_async_copy(src_hbm.at[pl.ds(0, CHUNKA)], src_v[k6], isem[k6]).wait()
      pltpu.make_async_copy(dst_hbm.at[pl.ds(0, CHUNKA)], dst_v[k6], isem[k6]).wait()
      pltpu.make_async_copy(ew_hbm.at[pl.ds(0, CHUNKA)], ew_v[k6], isem[k6]).wait()

    def gather(k6, k3):
      pltpu.async_copy(g_hbm.at[src_v[k6]], rows[k3], gsem[k3])

    def gwait(k3):
      pltpu.make_async_copy(g_hbm.at[src_v[0]], rows[k3], gsem[k3]).wait()

    def swait(k3):
      pltpu.make_async_copy(rows[0], acc.at[dst_v[0]], ssem[k3]).wait()

    for k in range(4):
      ifetch(k, k)

    def zrow(r, carry):
      for f in range(D // 16):
        rows[2][r, pl.ds(f * 16, 16)] = jnp.zeros((16,), jnp.float32)
      return carry

    lax.fori_loop(0, CHUNKA, zrow, 0)
    for j in range(RPT // CHUNKA):
      pltpu.sync_copy(rows[2], acc.at[pl.ds(s * RPT + j * CHUNKA, CHUNKA)])
    pltpu.sync_copy(rows[2].at[pl.ds(0, RPT % CHUNKA)],
                    acc.at[pl.ds(s * RPT + RPT - RPT % CHUNKA, RPT % CHUNKA)])
    plsc.subcore_barrier()

    iwait(0)
    gather(0, 0)
    iwait(1)
    gather(1, 1)
    pltpu.async_copy(rows[2], acc.at[dst_v[0]], ssem[2], add=True)

    def scale(k3, k6):
      r = rows[k3]

      def scale_m(m, c2):
        ew16 = ew_v[k6][pl.ds(m * 16, 16)]
        for t in range(16):
          w = ew16[t]
          e = m * 16 + t
          for f in range(D // 16):
            r[e, pl.ds(f * 16, 16)] = r[e, pl.ds(f * 16, 16)] * w
        return c2

      lax.fori_loop(0, CHUNKA // 16, scale_m, 0)

    def emit(j, k3, k6):
      gwait(k3)
      scale(k3, k6)
      pltpu.async_copy(rows[k3], acc.at[dst_v[k6]], ssem[k3], add=True)
      swait((k3 + 2) % 3)
      ifetch(j + 4, (k6 + 4) % 6)
      iwait((k6 + 2) % 6)
      gather((k6 + 2) % 6, (k3 + 2) % 3)

    def body(q, carry):
      j0 = 6 * q
      for k in range(6):
        emit(j0 + k, k % 3, k)
      return carry

    lax.fori_loop(0, NCHP // 6, body, 0)
    gwait(0)
    gwait(1)
    swait((NCHP - 1) % 3)
    iwait(2)
    iwait(3)

    plsc.subcore_barrier()
    pltpu.sync_copy(acc.at[pl.ds(s * RPT, RPT)],
                    part_out.at[c, pl.ds(s * RPT, RPT)])

  return sc_agg


def _dinv_body(dp_ref, out_ref):
  deg = dp_ref[0] + dp_ref[1] + 1.0
  out_ref[...] = jnp.where(deg > 0, 1.0 / jnp.sqrt(deg), 0.0)


def _tc_dinv(deg_pair):
  dp = deg_pair.reshape(NC, DEGP // D, D)
  out = pl.pallas_call(
      _dinv_body,
      out_shape=jax.ShapeDtypeStruct((DEGP // D, D), jnp.float32),
  )(dp)
  return out.reshape(DEGP, 1)[:N]


BLK = 1000
GRID = N // BLK


def _pre_body(x_ref, w_ref, dinv_ref, g_ref):
  h = jnp.dot(x_ref[...], w_ref[...], preferred_element_type=jnp.float32)
  g_ref[...] = h * dinv_ref[...]


def _tc_pre(x, W, dinv):
  return pl.pallas_call(
      _pre_body,
      grid=(GRID,),
      in_specs=[
          pl.BlockSpec((BLK, D), lambda i: (i, 0)),
          pl.BlockSpec((D, D), lambda i: (0, 0)),
          pl.BlockSpec((BLK, 1), lambda i: (i, 0)),
      ],
      out_specs=pl.BlockSpec((BLK, D), lambda i: (i, 0)),
      out_shape=jax.ShapeDtypeStruct((N, D), jnp.float32),
  )(x, W, dinv)


def _mid_body(p0_ref, p1_ref, g_ref, dinv_ref, b_ref, w_ref, out_ref):
  a = jax.nn.relu(dinv_ref[...] * (p0_ref[...] + p1_ref[...] + g_ref[...])
                  + b_ref[...])
  h = jnp.dot(a, w_ref[...], preferred_element_type=jnp.float32)
  out_ref[...] = h * dinv_ref[...]


def _tc_mid(part, g, dinv, b, Wn):
  return pl.pallas_call(
      _mid_body,
      grid=(GRID,),
      in_specs=[
          pl.BlockSpec((BLK, D), lambda i: (i, 0)),
          pl.BlockSpec((BLK, D), lambda i: (i, 0)),
          pl.BlockSpec((BLK, D), lambda i: (i, 0)),
          pl.BlockSpec((BLK, 1), lambda i: (i, 0)),
          pl.BlockSpec((1, D), lambda i: (0, 0)),
          pl.BlockSpec((D, D), lambda i: (0, 0)),
      ],
      out_specs=pl.BlockSpec((BLK, D), lambda i: (i, 0)),
      out_shape=jax.ShapeDtypeStruct((N, D), jnp.float32),
  )(part[0], part[1], g, dinv, b.reshape(1, D), Wn)


def _fin_body(p0_ref, p1_ref, g_ref, dinv_ref, b_ref, out_ref):
  out_ref[...] = jax.nn.sigmoid(
      dinv_ref[...] * (p0_ref[...] + p1_ref[...] + g_ref[...]) + b_ref[...])


def _tc_fin(part, g, dinv, b):
  return pl.pallas_call(
      _fin_body,
      grid=(GRID,),
      in_specs=[
          pl.BlockSpec((BLK, D), lambda i: (i, 0)),
          pl.BlockSpec((BLK, D), lambda i: (i, 0)),
          pl.BlockSpec((BLK, D), lambda i: (i, 0)),
          pl.BlockSpec((BLK, 1), lambda i: (i, 0)),
          pl.BlockSpec((1, D), lambda i: (0, 0)),
      ],
      out_specs=pl.BlockSpec((BLK, D), lambda i: (i, 0)),
      out_shape=jax.ShapeDtypeStruct((N, D), jnp.float32),
  )(part[0], part[1], g, dinv, b.reshape(1, D))


@jax.jit
def kernel(x, edge_index, edge_weight, W1, b1, W2, b2, W3, b3, W4, b4):
  src = edge_index[0]
  dst = edge_index[1]
  zpad_i = jnp.zeros((NW, EPTP - EPT), jnp.int32)
  zpad_f = jnp.zeros((NW, EPTP - EPT), jnp.float32)
  srcp = jnp.concatenate([src.reshape(NW, EPT), zpad_i], 1).reshape(-1)
  dstp = jnp.concatenate([dst.reshape(NW, EPT), zpad_i], 1).reshape(-1)
  ewp = jnp.concatenate([edge_weight.reshape(NW, EPT), zpad_f], 1).reshape(-1)


  deg_pair = _build_sc_degree()(dst, edge_weight)
  dinv = _tc_dinv(deg_pair)

  sc_agg = _build_sc_agg()
  g = _tc_pre(x, W1, dinv)
  part = sc_agg(g, srcp, dstp, ewp)
  g = _tc_mid(part, g, dinv, b1, W2)
  part = sc_agg(g, srcp, dstp, ewp)
  g = _tc_mid(part, g, dinv, b2, W3)
  part = sc_agg(g, srcp, dstp, ewp)
  g = _tc_mid(part, g, dinv, b3, W4)
  part = sc_agg(g, srcp, dstp, ewp)
  return _tc_fin(part, g, dinv, b4)


# speedup vs baseline: 2.8110x; 2.8110x over previous
import functools
import jax
import jax.numpy as jnp
from jax import lax
from jax.experimental import pallas as pl
from jax.experimental.pallas import tpu as pltpu
from jax.experimental.pallas import tpu_sc as plsc

N = 10000
E = 320000
D = 128

NC = 2
NS = 16
NW = NC * NS
EPT = E // NW
CHUNK = 80
NCHUNK = EPT // CHUNK
NP = 10240
RPT = NP // NS
ZR = RPT // 5
DEGP = 10240
DPT = DEGP // NS


@functools.cache
def _build_sc_degree():
  mesh = plsc.VectorSubcoreMesh(core_axis_name="c", subcore_axis_name="s")

  @functools.partial(
      pl.kernel,
      out_type=jax.ShapeDtypeStruct((NC, DEGP), jnp.float32),
      mesh=mesh,
      scratch_types=[
          pltpu.MemorySpace.VMEM_SHARED((DEGP,), jnp.float32),
          pltpu.MemorySpace.VMEM((DPT,), jnp.float32),
          [pltpu.MemorySpace.VMEM((CHUNK,), jnp.int32) for _ in range(8)],
          [pltpu.MemorySpace.VMEM((CHUNK,), jnp.float32) for _ in range(8)],
          [pltpu.SemaphoreType.DMA for _ in range(8)],
          [pltpu.SemaphoreType.DMA for _ in range(4)],
      ],
  )
  def sc_degree(dst_hbm, ew_hbm, deg_out, dacc, zv, dst_v, ew_v, isem, ssem):
    c = lax.axis_index("c")
    s = lax.axis_index("s")
    wid = c * NS + s
    base = wid * EPT

    def ifetch(j, k8):
      off = base + j * CHUNK
      pltpu.async_copy(dst_hbm.at[pl.ds(off, CHUNK)], dst_v[k8], isem[k8])
      pltpu.async_copy(ew_hbm.at[pl.ds(off, CHUNK)], ew_v[k8], isem[k8])

    def iwait(k8):
      pltpu.make_async_copy(dst_hbm.at[pl.ds(0, CHUNK)], dst_v[k8], isem[k8]).wait()
      pltpu.make_async_copy(ew_hbm.at[pl.ds(0, CHUNK)], ew_v[k8], isem[k8]).wait()

    def swait(k4):
      pltpu.make_async_copy(ew_v[0], dacc.at[dst_v[0]], ssem[k4]).wait()

    for k in range(6):
      ifetch(k, k)

    for j in range(DPT // 16):
      zv[pl.ds(j * 16, 16)] = jnp.zeros((16,), jnp.float32)
    pltpu.sync_copy(zv, dacc.at[pl.ds(s * DPT, DPT)])
    plsc.subcore_barrier()

    def emit(j, k4, k8, has_swait, has_fetch):
      iwait(k8)
      pltpu.async_copy(ew_v[k8], dacc.at[dst_v[k8]], ssem[k4], add=True)
      if has_swait:
        swait((k4 + 2) % 4)
      if has_fetch:
        ifetch(j + 6, (k8 + 6) % 8)

    for j in range(8):
      emit(j, j % 4, j % 8, j >= 2, True)

    def body(q, carry):
      j0 = 8 * q
      for k in range(8):
        emit(j0 + k, k % 4, k, True, True)
      return carry

    lax.fori_loop(1, 14, body, 0)
    for j in range(112, NCHUNK):
      emit(j, j % 4, j % 8, True, j + 6 < NCHUNK)
    swait((NCHUNK - 2) % 4)
    swait((NCHUNK - 1) % 4)

    plsc.subcore_barrier()
    pltpu.sync_copy(dacc.at[pl.ds(s * DPT, DPT)],
                    deg_out.at[c, pl.ds(s * DPT, DPT)])

  return sc_degree


@functools.cache
def _build_sc_agg():
  mesh = plsc.VectorSubcoreMesh(core_axis_name="c", subcore_axis_name="s")

  @functools.partial(
      pl.kernel,
      out_type=jax.ShapeDtypeStruct((NC, NP, D), jnp.float32),
      mesh=mesh,
      scratch_types=[
          pltpu.MemorySpace.VMEM_SHARED((NP, D), jnp.float32),
          [pltpu.MemorySpace.VMEM((CHUNK, D), jnp.float32) for _ in range(4)],
          [pltpu.MemorySpace.VMEM((CHUNK,), jnp.int32) for _ in range(8)],
          [pltpu.MemorySpace.VMEM((CHUNK,), jnp.int32) for _ in range(8)],
          [pltpu.MemorySpace.VMEM((CHUNK,), jnp.float32) for _ in range(8)],
          [pltpu.SemaphoreType.DMA for _ in range(4)],
          [pltpu.SemaphoreType.DMA for _ in range(4)],
          [pltpu.SemaphoreType.DMA for _ in range(8)],
      ],
  )
  def sc_agg(g_hbm, src_hbm, dst_hbm, ew_hbm, part_out,
             acc, rows, src_v, dst_v, ew_v, gsem, ssem, isem):
    c = lax.axis_index("c")
    s = lax.axis_index("s")
    wid = c * NS + s
    base = wid * EPT

    def ifetch(j, k8):
      off = base + jnp.minimum(j, NCHUNK - 1) * CHUNK
      pltpu.async_copy(src_hbm.at[pl.ds(off, CHUNK)], src_v[k8], isem[k8])
      pltpu.async_copy(dst_hbm.at[pl.ds(off, CHUNK)], dst_v[k8], isem[k8])
      pltpu.async_copy(ew_hbm.at[pl.ds(off, CHUNK)], ew_v[k8], isem[k8])

    def iwait(k8):
      pltpu.make_async_copy(src_hbm.at[pl.ds(0, CHUNK)], src_v[k8], isem[k8]).wait()
      pltpu.make_async_copy(dst_hbm.at[pl.ds(0, CHUNK)], dst_v[k8], isem[k8]).wait()
      pltpu.make_async_copy(ew_hbm.at[pl.ds(0, CHUNK)], ew_v[k8], isem[k8]).wait()

    def gather(k8, k4):
      pltpu.async_copy(g_hbm.at[src_v[k8]], rows[k4], gsem[k4])

    def gwait(k4):
      pltpu.make_async_copy(g_hbm.at[src_v[0]], rows[k4], gsem[k4]).wait()

    def swait(k4):
      pltpu.make_async_copy(rows[0], acc.at[dst_v[0]], ssem[k4]).wait()

    for k in range(6):
      ifetch(k, k)

    def zrow(r, carry):
      for f in range(D // 16):
        rows[3][r, pl.ds(f * 16, 16)] = jnp.zeros((16,), jnp.float32)
      return carry

    lax.fori_loop(0, CHUNK, zrow, 0)
    for j in range(RPT // CHUNK):
      pltpu.sync_copy(rows[3], acc.at[pl.ds(s * RPT + j * CHUNK, CHUNK)])
    plsc.subcore_barrier()

    iwait(0)
    gather(0, 0)
    iwait(1)
    gather(1, 1)
    pltpu.async_copy(rows[3], acc.at[dst_v[0]], ssem[2], add=True)
    pltpu.async_copy(rows[3], acc.at[dst_v[0]], ssem[3], add=True)

    def scale(k4, k8):
      r = rows[k4]

      def scale_m(m, c2):
        ew16 = ew_v[k8][pl.ds(m * 16, 16)]
        for t in range(16):
          w = ew16[t]
          e = m * 16 + t
          for f in range(D // 16):
            r[e, pl.ds(f * 16, 16)] = r[e, pl.ds(f * 16, 16)] * w
        return c2

      lax.fori_loop(0, CHUNK // 16, scale_m, 0)

    def emit(j, k4, k8):
      gwait(k4)
      swait((k4 + 2) % 4)
      ifetch(j + 6, (k8 + 6) % 8)
      iwait((k8 + 2) % 8)
      gather((k8 + 2) % 8, (k4 + 2) % 4)
      scale(k4, k8)
      pltpu.async_copy(rows[k4], acc.at[dst_v[k8]], ssem[k4], add=True)

    def body(q, carry):
      j0 = 8 * q
      for k in range(8):
        emit(j0 + k, k % 4, k)
      return carry

    lax.fori_loop(0, 15, body, 0)
    for j in range(120, NCHUNK):
      emit(j, j % 4, j % 8)
    gwait(1)
    gwait(2)
    swait((NCHUNK - 2) % 4)
    swait((NCHUNK - 1) % 4)
    for st in (7, 0, 1, 2):
      iwait(st)

    plsc.subcore_barrier()
    pltpu.sync_copy(acc.at[pl.ds(s * RPT, RPT)],
                    part_out.at[c, pl.ds(s * RPT, RPT)])

  return sc_agg


def _dinv_body(dp_ref, out_ref):
  deg = dp_ref[0] + dp_ref[1] + 1.0
  out_ref[...] = jnp.where(deg > 0, 1.0 / jnp.sqrt(deg), 0.0)


def _tc_dinv(deg_pair):
  dp = deg_pair.reshape(NC, DEGP // D, D)
  out = pl.pallas_call(
      _dinv_body,
      out_shape=jax.ShapeDtypeStruct((DEGP // D, D), jnp.float32),
  )(dp)
  return out.reshape(DEGP, 1)[:N]


BLK = 1000
GRID = N // BLK


def _pre_body(x_ref, w_ref, dinv_ref, g_ref):
  h = jnp.dot(x_ref[...], w_ref[...], preferred_element_type=jnp.float32)
  g_ref[...] = h * dinv_ref[...]


def _tc_pre(x, W, dinv):
  return pl.pallas_call(
      _pre_body,
      grid=(GRID,),
      in_specs=[
          pl.BlockSpec((BLK, D), lambda i: (i, 0)),
          pl.BlockSpec((D, D), lambda i: (0, 0)),
          pl.BlockSpec((BLK, 1), lambda i: (i, 0)),
      ],
      out_specs=pl.BlockSpec((BLK, D), lambda i: (i, 0)),
      out_shape=jax.ShapeDtypeStruct((N, D), jnp.float32),
  )(x, W, dinv)


def _mid_body(p0_ref, p1_ref, g_ref, dinv_ref, b_ref, w_ref, out_ref):
  a = jax.nn.relu(dinv_ref[...] * (p0_ref[...] + p1_ref[...] + g_ref[...])
                  + b_ref[...])
  h = jnp.dot(a, w_ref[...], preferred_element_type=jnp.float32)
  out_ref[...] = h * dinv_ref[...]


def _tc_mid(part, g, dinv, b, Wn):
  return pl.pallas_call(
      _mid_body,
      grid=(GRID,),
      in_specs=[
          pl.BlockSpec((BLK, D), lambda i: (i, 0)),
          pl.BlockSpec((BLK, D), lambda i: (i, 0)),
          pl.BlockSpec((BLK, D), lambda i: (i, 0)),
          pl.BlockSpec((BLK, 1), lambda i: (i, 0)),
          pl.BlockSpec((1, D), lambda i: (0, 0)),
          pl.BlockSpec((D, D), lambda i: (0, 0)),
      ],
      out_specs=pl.BlockSpec((BLK, D), lambda i: (i, 0)),
      out_shape=jax.ShapeDtypeStruct((N, D), jnp.float32),
  )(part[0], part[1], g, dinv, b.reshape(1, D), Wn)


def _fin_body(p0_ref, p1_ref, g_ref, dinv_ref, b_ref, out_ref):
  out_ref[...] = jax.nn.sigmoid(
      dinv_ref[...] * (p0_ref[...] + p1_ref[...] + g_ref[...]) + b_ref[...])


def _tc_fin(part, g, dinv, b):
  return pl.pallas_call(
      _fin_body,
      grid=(GRID,),
      in_specs=[
          pl.BlockSpec((BLK, D), lambda i: (i, 0)),
          pl.BlockSpec((BLK, D), lambda i: (i, 0)),
          pl.BlockSpec((BLK, D), lambda i: (i, 0)),
          pl.BlockSpec((BLK, 1), lambda i: (i, 0)),
          pl.BlockSpec((1, D), lambda i: (0, 0)),
      ],
      out_specs=pl.BlockSpec((BLK, D), lambda i: (i, 0)),
      out_shape=jax.ShapeDtypeStruct((N, D), jnp.float32),
  )(part[0], part[1], g, dinv, b.reshape(1, D))


@jax.jit
def kernel(x, edge_index, edge_weight, W1, b1, W2, b2, W3, b3, W4, b4):
  src = edge_index[0]
  dst = edge_index[1]


  deg_pair = _build_sc_degree()(dst, edge_weight)
  dinv = _tc_dinv(deg_pair)

  sc_agg = _build_sc_agg()
  g = _tc_pre(x, W1, dinv)
  part = sc_agg(g, src, dst, edge_weight)
  g = _tc_mid(part, g, dinv, b1, W2)
  part = sc_agg(g, src, dst, edge_weight)
  g = _tc_mid(part, g, dinv, b2, W3)
  part = sc_agg(g, src, dst, edge_weight)
  g = _tc_mid(part, g, dinv, b3, W4)
  part = sc_agg(g, src, dst, edge_weight)
  return _tc_fin(part, g, dinv, b4)
